# Initial kernel scaffold; baseline (speedup 1.0000x reference)
import functools

import jax
import jax.numpy as jnp
from jax import lax
from jax.experimental import pallas as pl
from jax.experimental.pallas import tpu as pltpu
from jax.experimental.pallas import tpu_sc as plsc

N_NODES_K = 10000
D = 128
NC = 2
NS = 16
NW = NC * NS
CHUNK = 128
NPAD = 10240
ROWS_PER_TILE = NPAD // NS


def _ceil_to(a, m):
    return (a + m - 1) // m * m


def _sc_segment_sum(x, src, dst, w):
    E = src.shape[0]
    e_per_w = E // NW
    rounds = e_per_w // CHUNK
    mesh = plsc.VectorSubcoreMesh(core_axis_name="c", subcore_axis_name="s")

    @functools.partial(
        pl.kernel,
        out_type=jax.ShapeDtypeStruct((NC, NPAD, D), jnp.float32),
        mesh=mesh,
        scratch_types=[
            pltpu.VMEM((CHUNK,), jnp.int32),
            pltpu.VMEM((CHUNK,), jnp.int32),
            pltpu.VMEM((CHUNK,), jnp.float32),
            pltpu.VMEM((CHUNK, D), jnp.float32),
            pltpu.VMEM_SHARED((NPAD, D), jnp.float32),
            pltpu.SemaphoreType.DMA,
        ],
    )
    def k(x_hbm, src_hbm, dst_hbm, w_hbm, zero_hbm, out_hbm,
          src_v, dst_v, w_v, rows_v, agg_sh, sem):
        c = lax.axis_index("c")
        s = lax.axis_index("s")
        wid = c * NS + s
        stripe = pl.ds(s * ROWS_PER_TILE, ROWS_PER_TILE)
        pltpu.sync_copy(zero_hbm.at[stripe], agg_sh.at[stripe])
        plsc.subcore_barrier()

        base = wid * e_per_w

        def round_body(r, carry):
            off = base + r * CHUNK
            pltpu.sync_copy(src_hbm.at[pl.ds(off, CHUNK)], src_v)
            pltpu.sync_copy(dst_hbm.at[pl.ds(off, CHUNK)], dst_v)
            pltpu.sync_copy(w_hbm.at[pl.ds(off, CHUNK)], w_v)
            pltpu.async_copy(x_hbm.at[src_v], rows_v, sem).wait()

            def mul_body(e, c2):
                we = w_v[e]
                for j in range(D // 16):
                    sl = pl.ds(j * 16, 16)
                    rows_v[e, sl] = rows_v[e, sl] * we
                return c2

            lax.fori_loop(0, CHUNK, mul_body, 0, unroll=2)
            pltpu.sync_copy(rows_v, agg_sh.at[dst_v], add=True)
            return carry

        lax.fori_loop(0, rounds, round_body, 0)
        plsc.subcore_barrier()
        pltpu.sync_copy(agg_sh.at[stripe], out_hbm.at[c].at[stripe])

    zeros = jnp.zeros((NPAD, D), jnp.float32)
    return k(x, src, dst, w, zeros)


def _tc_linear(p0, p1, wt, b2):
    blk = 1000
    grid = (N_NODES_K // blk,)

    def body(p0_ref, p1_ref, wt_ref, b_ref, out_ref):
        agg = p0_ref[...] + p1_ref[...]
        out_ref[...] = jnp.dot(
            agg, wt_ref[...], preferred_element_type=jnp.float32
        ) + b_ref[...]

    return pl.pallas_call(
        body,
        out_shape=jax.ShapeDtypeStruct((N_NODES_K, D), jnp.float32),
        grid=grid,
        in_specs=[
            pl.BlockSpec((blk, D), lambda i: (i, 0)),
            pl.BlockSpec((blk, D), lambda i: (i, 0)),
            pl.BlockSpec((D, D), lambda i: (0, 0)),
            pl.BlockSpec((1, D), lambda i: (0, 0)),
        ],
        out_specs=pl.BlockSpec((blk, D), lambda i: (i, 0)),
    )(p0, p1, wt, b2)


def kernel(x, edge_index, edge_weight, W, b):
    dst = edge_index[0].astype(jnp.int32)
    src = edge_index[1].astype(jnp.int32)
    w = edge_weight.astype(jnp.float32)
    e0 = src.shape[0]
    e_pad = _ceil_to(e0, NW * CHUNK)
    pad = e_pad - e0
    if pad:
        src = jnp.concatenate([src, jnp.zeros((pad,), jnp.int32)])
        dst = jnp.concatenate([dst, jnp.zeros((pad,), jnp.int32)])
        w = jnp.concatenate([w, jnp.zeros((pad,), jnp.float32)])
    p = _sc_segment_sum(x, src, dst, w)
    return _tc_linear(p[0], p[1], W.T, b.reshape(1, D))


# speedup vs baseline: 4.1084x; 4.1084x over previous
import functools

import jax
import jax.numpy as jnp
from jax import lax
from jax.experimental import pallas as pl
from jax.experimental.pallas import tpu as pltpu
from jax.experimental.pallas import tpu_sc as plsc

N_NODES_K = 10000
D = 128
NC = 2
NS = 16
NW = NC * NS
CHUNK = 128
NPAD = 10240
ROWS_PER_TILE = NPAD // NS


def _ceil_to(a, m):
    return (a + m - 1) // m * m


def _sc_segment_sum(x, src, dst, w):
    E = src.shape[0]
    e_per_w = E // NW
    rounds = e_per_w // CHUNK
    mesh = plsc.VectorSubcoreMesh(core_axis_name="c", subcore_axis_name="s")

    @functools.partial(
        pl.kernel,
        out_type=jax.ShapeDtypeStruct((NC, NPAD, D), jnp.float32),
        mesh=mesh,
        scratch_types=[
            pltpu.VMEM((CHUNK,), jnp.int32),
            pltpu.VMEM((CHUNK,), jnp.int32),
            pltpu.VMEM((CHUNK,), jnp.float32),
            pltpu.VMEM((CHUNK, D), jnp.float32),
            pltpu.VMEM_SHARED((NPAD, D), jnp.float32),
            pltpu.SemaphoreType.DMA,
        ],
    )
    def k(x_hbm, src_hbm, dst_hbm, w_hbm, zero_hbm, out_hbm,
          src_v, dst_v, w_v, rows_v, agg_sh, sem):
        c = lax.axis_index("c")
        s = lax.axis_index("s")
        wid = c * NS + s
        stripe = pl.ds(s * ROWS_PER_TILE, ROWS_PER_TILE)
        pltpu.sync_copy(zero_hbm.at[stripe], agg_sh.at[stripe])
        plsc.subcore_barrier()

        base = wid * e_per_w

        def round_body(r, carry):
            off = base + r * CHUNK
            pltpu.sync_copy(src_hbm.at[pl.ds(off, CHUNK)], src_v)
            pltpu.sync_copy(dst_hbm.at[pl.ds(off, CHUNK)], dst_v)
            pltpu.sync_copy(w_hbm.at[pl.ds(off, CHUNK)], w_v)
            pltpu.async_copy(x_hbm.at[src_v], rows_v, sem).wait()

            def mul_body(g, c2):
                wvec = w_v[pl.ds(g * 16, 16)]
                for i in range(16):
                    we = wvec[i]
                    e = g * 16 + i
                    for j in range(D // 16):
                        sl = pl.ds(j * 16, 16)
                        rows_v[e, sl] = rows_v[e, sl] * we
                return c2

            lax.fori_loop(0, CHUNK // 16, mul_body, 0)
            pltpu.sync_copy(rows_v, agg_sh.at[dst_v], add=True)
            return carry

        lax.fori_loop(0, rounds, round_body, 0)
        plsc.subcore_barrier()
        pltpu.sync_copy(agg_sh.at[stripe], out_hbm.at[c].at[stripe])

    zeros = jnp.zeros((NPAD, D), jnp.float32)
    return k(x, src, dst, w, zeros)


def _tc_linear(p0, p1, wt, b2):
    blk = 1000
    grid = (N_NODES_K // blk,)

    def body(p0_ref, p1_ref, wt_ref, b_ref, out_ref):
        agg = p0_ref[...] + p1_ref[...]
        out_ref[...] = jnp.dot(
            agg, wt_ref[...], preferred_element_type=jnp.float32
        ) + b_ref[...]

    return pl.pallas_call(
        body,
        out_shape=jax.ShapeDtypeStruct((N_NODES_K, D), jnp.float32),
        grid=grid,
        in_specs=[
            pl.BlockSpec((blk, D), lambda i: (i, 0)),
            pl.BlockSpec((blk, D), lambda i: (i, 0)),
            pl.BlockSpec((D, D), lambda i: (0, 0)),
            pl.BlockSpec((1, D), lambda i: (0, 0)),
        ],
        out_specs=pl.BlockSpec((blk, D), lambda i: (i, 0)),
    )(p0, p1, wt, b2)


def kernel(x, edge_index, edge_weight, W, b):
    dst = edge_index[0].astype(jnp.int32)
    src = edge_index[1].astype(jnp.int32)
    w = edge_weight.astype(jnp.float32)
    e0 = src.shape[0]
    e_pad = _ceil_to(e0, NW * CHUNK)
    pad = e_pad - e0
    if pad:
        src = jnp.concatenate([src, jnp.zeros((pad,), jnp.int32)])
        dst = jnp.concatenate([dst, jnp.zeros((pad,), jnp.int32)])
        w = jnp.concatenate([w, jnp.zeros((pad,), jnp.float32)])
    p = _sc_segment_sum(x, src, dst, w)
    return _tc_linear(p[0], p[1], W.T, b.reshape(1, D))
